# baseline (speedup 1.0000x reference)
import jax
import jax.numpy as jnp
import numpy as np
from jax import lax
from jax.experimental import pallas as pl
from jax.experimental.pallas import tpu as pltpu
from jax.experimental.pallas import tpu_sc as plsc

D_MODEL = 64
MAX_LEN = 200
SEQ_LEN = 200
BATCH = 1024

NUM_CORES = 2
NUM_SUBCORES = 16
NUM_WORKERS = NUM_CORES * NUM_SUBCORES

BT = BATCH * SEQ_LEN
ROWS_PER_W = BT // NUM_WORKERS
CHUNK = 128
NCHUNKS = ROWS_PER_W // CHUNK
PE_EXT = SEQ_LEN + CHUNK


def _positional_encoding_ext() -> np.ndarray:
    pe = np.zeros((MAX_LEN, D_MODEL), dtype=np.float32)
    pos = np.arange(0, MAX_LEN, dtype=np.float32)[:, None]
    _2i = np.arange(0, D_MODEL, 2, dtype=np.float32)
    pe[:, 0::2] = np.sin(pos / (10000.0 ** (_2i / D_MODEL)))
    pe[:, 1::2] = np.cos(pos / (10000.0 ** (_2i / D_MODEL)))
    return np.concatenate([pe, pe[:CHUNK]], axis=0)


_PE_EXT_CONST = _positional_encoding_ext()


def _body(x_hbm, pe_hbm, table_hbm, out_hbm, idx_v, pe_v, rows_v, gsem):
    wid = lax.axis_index("s") * NUM_CORES + lax.axis_index("c")
    base = wid * ROWS_PER_W

    pltpu.sync_copy(x_hbm.at[pl.ds(base, ROWS_PER_W)], idx_v)
    pltpu.sync_copy(pe_hbm, pe_v)

    @pl.loop(0, NCHUNKS)
    def _chunk(c):
        row0 = c * CHUNK
        pltpu.async_copy(table_hbm.at[idx_v.at[pl.ds(row0, CHUNK)]],
                         rows_v, gsem).wait()
        s0 = lax.rem(row0, SEQ_LEN)

        @pl.loop(0, CHUNK, unroll=4)
        def _row(i):
            for j in range(D_MODEL // 16):
                sl = pl.ds(j * 16, 16)
                rows_v[i, sl] += pe_v[s0 + i, sl]

        pltpu.sync_copy(rows_v, out_hbm.at[pl.ds(base + row0, CHUNK)])


@jax.jit
def _embed(x_flat, pe_ext, table):
    kfn = pl.kernel(
        _body,
        out_type=jax.ShapeDtypeStruct((BT, D_MODEL), jnp.float32),
        mesh=plsc.VectorSubcoreMesh(core_axis_name="c", subcore_axis_name="s"),
        scratch_types=[
            pltpu.VMEM((ROWS_PER_W,), jnp.int32),
            pltpu.VMEM((PE_EXT, D_MODEL), jnp.float32),
            pltpu.VMEM((CHUNK, D_MODEL), jnp.float32),
            pltpu.SemaphoreType.DMA,
        ],
        compiler_params=pltpu.CompilerParams(use_tc_tiling_on_sc=False),
    )
    return kfn(x_flat, pe_ext, table)


def kernel(x, table):
    x_flat = x.reshape(BT).astype(jnp.int32)
    pe_ext = jnp.asarray(_PE_EXT_CONST)
    out = _embed(x_flat, pe_ext, table)
    return out.reshape(BATCH, SEQ_LEN, D_MODEL)


# speedup vs baseline: 1.0652x; 1.0652x over previous
import jax
import jax.numpy as jnp
import numpy as np
from jax import lax
from jax.experimental import pallas as pl
from jax.experimental.pallas import tpu as pltpu
from jax.experimental.pallas import tpu_sc as plsc

D_MODEL = 64
MAX_LEN = 200
SEQ_LEN = 200
BATCH = 1024

NUM_CORES = 2
NUM_SUBCORES = 16
NUM_WORKERS = NUM_CORES * NUM_SUBCORES

BT = BATCH * SEQ_LEN
ROWS_PER_W = BT // NUM_WORKERS
CHUNK = 128
NCHUNKS = ROWS_PER_W // CHUNK
PE_EXT = SEQ_LEN + CHUNK


def _positional_encoding_ext() -> np.ndarray:
    pe = np.zeros((MAX_LEN, D_MODEL), dtype=np.float32)
    pos = np.arange(0, MAX_LEN, dtype=np.float32)[:, None]
    _2i = np.arange(0, D_MODEL, 2, dtype=np.float32)
    pe[:, 0::2] = np.sin(pos / (10000.0 ** (_2i / D_MODEL)))
    pe[:, 1::2] = np.cos(pos / (10000.0 ** (_2i / D_MODEL)))
    return np.concatenate([pe, pe[:CHUNK]], axis=0)


_PE_EXT_CONST = _positional_encoding_ext()


NBUF = 5
LOOK = 3


def _body(x_hbm, pe_hbm, table_hbm, out_hbm, idx_v, pe_v, rows_v, gsem, ssem):
    wid = lax.axis_index("s") * NUM_CORES + lax.axis_index("c")
    base = wid * ROWS_PER_W

    pltpu.sync_copy(x_hbm.at[pl.ds(base, ROWS_PER_W)], idx_v)
    pltpu.sync_copy(pe_hbm, pe_v)

    def start_gather(c, b):
        pltpu.async_copy(
            table_hbm.at[idx_v.at[pl.ds(c * CHUNK, CHUNK)]],
            rows_v.at[b], gsem.at[b])

    def wait_gather(b):
        pltpu.make_async_copy(
            table_hbm.at[idx_v.at[pl.ds(0, CHUNK)]],
            rows_v.at[b], gsem.at[b]).wait()

    def start_store(c, b):
        pltpu.async_copy(
            rows_v.at[b], out_hbm.at[pl.ds(base + c * CHUNK, CHUNK)],
            ssem.at[b])

    def wait_store(b):
        pltpu.make_async_copy(
            rows_v.at[b], out_hbm.at[pl.ds(base, CHUNK)], ssem.at[b]).wait()

    for c in range(LOOK):
        start_gather(c, c)

    @pl.loop(0, NCHUNKS // NBUF)
    def _group(g):
        for b in range(NBUF):
            c = g * NBUF + b
            b2 = (b + LOOK) % NBUF

            @pl.when(c + LOOK < NCHUNKS)
            def _():
                @pl.when(c + LOOK >= NBUF)
                def _():
                    wait_store(b2)
                start_gather(c + LOOK, b2)

            wait_gather(b)
            s0 = lax.rem(c * CHUNK, SEQ_LEN)

            @pl.loop(0, CHUNK, unroll=4)
            def _row(i):
                for j in range(D_MODEL // 16):
                    sl = pl.ds(j * 16, 16)
                    rows_v[b, i, sl] += pe_v[s0 + i, sl]

            start_store(c, b)

    for b in range(NBUF):
        wait_store(b)


@jax.jit
def _embed(x_flat, pe_ext, table):
    kfn = pl.kernel(
        _body,
        out_type=jax.ShapeDtypeStruct((BT, D_MODEL), jnp.float32),
        mesh=plsc.VectorSubcoreMesh(core_axis_name="c", subcore_axis_name="s"),
        scratch_types=[
            pltpu.VMEM((ROWS_PER_W,), jnp.int32),
            pltpu.VMEM((PE_EXT, D_MODEL), jnp.float32),
            pltpu.VMEM((NBUF, CHUNK, D_MODEL), jnp.float32),
            pltpu.SemaphoreType.DMA((NBUF,)),
            pltpu.SemaphoreType.DMA((NBUF,)),
        ],
        compiler_params=pltpu.CompilerParams(use_tc_tiling_on_sc=False),
    )
    return kfn(x_flat, pe_ext, table)


def kernel(x, table):
    x_flat = x.reshape(BT).astype(jnp.int32)
    pe_ext = jnp.asarray(_PE_EXT_CONST)
    out = _embed(x_flat, pe_ext, table)
    return out.reshape(BATCH, SEQ_LEN, D_MODEL)


# speedup vs baseline: 1.0667x; 1.0014x over previous
import jax
import jax.numpy as jnp
import numpy as np
from jax import lax
from jax.experimental import pallas as pl
from jax.experimental.pallas import tpu as pltpu
from jax.experimental.pallas import tpu_sc as plsc

D_MODEL = 64
MAX_LEN = 200
SEQ_LEN = 200
BATCH = 1024

NUM_CORES = 2
NUM_SUBCORES = 16
NUM_WORKERS = NUM_CORES * NUM_SUBCORES

BT = BATCH * SEQ_LEN
ROWS_PER_W = BT // NUM_WORKERS
CHUNK = 128
NCHUNKS = ROWS_PER_W // CHUNK
PE_EXT = SEQ_LEN + CHUNK


def _positional_encoding_ext() -> np.ndarray:
    pe = np.zeros((MAX_LEN, D_MODEL), dtype=np.float32)
    pos = np.arange(0, MAX_LEN, dtype=np.float32)[:, None]
    _2i = np.arange(0, D_MODEL, 2, dtype=np.float32)
    pe[:, 0::2] = np.sin(pos / (10000.0 ** (_2i / D_MODEL)))
    pe[:, 1::2] = np.cos(pos / (10000.0 ** (_2i / D_MODEL)))
    return np.concatenate([pe, pe[:CHUNK]], axis=0)


_PE_EXT_CONST = _positional_encoding_ext()


NBUF = 5
LOOK = 3


SEQ_PER_W = BATCH // NUM_WORKERS


def _body(x_hbm, pe_hbm, table_hbm, out_hbm, idx_v, pe_v, rows_v, gsem, ssem,
          isem):
    wid = lax.axis_index("s") * NUM_CORES + lax.axis_index("c")
    base = wid * ROWS_PER_W
    seq0 = wid * SEQ_PER_W

    for s in range(SEQ_PER_W):
        pltpu.async_copy(x_hbm.at[seq0 + s],
                         idx_v.at[pl.ds(s * SEQ_LEN, SEQ_LEN)], isem)
    pltpu.sync_copy(pe_hbm, pe_v)
    for s in range(SEQ_PER_W):
        pltpu.make_async_copy(x_hbm.at[seq0],
                              idx_v.at[pl.ds(0, SEQ_LEN)], isem).wait()

    def start_gather(c, b):
        pltpu.async_copy(
            table_hbm.at[idx_v.at[pl.ds(c * CHUNK, CHUNK)]],
            rows_v.at[b], gsem.at[b])

    def wait_gather(b):
        pltpu.make_async_copy(
            table_hbm.at[idx_v.at[pl.ds(0, CHUNK)]],
            rows_v.at[b], gsem.at[b]).wait()

    def start_store(c, b):
        pltpu.async_copy(
            rows_v.at[b], out_hbm.at[pl.ds(base + c * CHUNK, CHUNK)],
            ssem.at[b])

    def wait_store(b):
        pltpu.make_async_copy(
            rows_v.at[b], out_hbm.at[pl.ds(base, CHUNK)], ssem.at[b]).wait()

    for c in range(LOOK):
        start_gather(c, c)

    @pl.loop(0, NCHUNKS // NBUF)
    def _group(g):
        for b in range(NBUF):
            c = g * NBUF + b
            b2 = (b + LOOK) % NBUF

            @pl.when(c + LOOK < NCHUNKS)
            def _():
                @pl.when(c + LOOK >= NBUF)
                def _():
                    wait_store(b2)
                start_gather(c + LOOK, b2)

            wait_gather(b)
            s0 = lax.rem(c * CHUNK, SEQ_LEN)

            @pl.loop(0, CHUNK, unroll=4)
            def _row(i):
                for j in range(D_MODEL // 16):
                    sl = pl.ds(j * 16, 16)
                    rows_v[b, i, sl] += pe_v[s0 + i, sl]

            start_store(c, b)

    for b in range(NBUF):
        wait_store(b)


@jax.jit
def _embed(x_flat, pe_ext, table):
    kfn = pl.kernel(
        _body,
        out_type=jax.ShapeDtypeStruct((BT, D_MODEL), jnp.float32),
        mesh=plsc.VectorSubcoreMesh(core_axis_name="c", subcore_axis_name="s"),
        scratch_types=[
            pltpu.VMEM((ROWS_PER_W,), jnp.int32),
            pltpu.VMEM((PE_EXT, D_MODEL), jnp.float32),
            pltpu.VMEM((NBUF, CHUNK, D_MODEL), jnp.float32),
            pltpu.SemaphoreType.DMA((NBUF,)),
            pltpu.SemaphoreType.DMA((NBUF,)),
            pltpu.SemaphoreType.DMA,
        ],
        compiler_params=pltpu.CompilerParams(use_tc_tiling_on_sc=False),
    )
    return kfn(x_flat, pe_ext, table)


def kernel(x, table):
    pe_ext = jnp.asarray(_PE_EXT_CONST)
    out = _embed(x.astype(jnp.int32), pe_ext, table)
    return out.reshape(BATCH, SEQ_LEN, D_MODEL)


# speedup vs baseline: 1.0673x; 1.0006x over previous
import jax
import jax.numpy as jnp
import numpy as np
from jax import lax
from jax.experimental import pallas as pl
from jax.experimental.pallas import tpu as pltpu
from jax.experimental.pallas import tpu_sc as plsc

D_MODEL = 64
MAX_LEN = 200
SEQ_LEN = 200
BATCH = 1024

NUM_CORES = 2
NUM_SUBCORES = 16
NUM_WORKERS = NUM_CORES * NUM_SUBCORES

BT = BATCH * SEQ_LEN
ROWS_PER_W = BT // NUM_WORKERS
CHUNK = 128
NCHUNKS = ROWS_PER_W // CHUNK
PE_EXT = SEQ_LEN + CHUNK


def _positional_encoding_ext() -> np.ndarray:
    pe = np.zeros((MAX_LEN, D_MODEL), dtype=np.float32)
    pos = np.arange(0, MAX_LEN, dtype=np.float32)[:, None]
    _2i = np.arange(0, D_MODEL, 2, dtype=np.float32)
    pe[:, 0::2] = np.sin(pos / (10000.0 ** (_2i / D_MODEL)))
    pe[:, 1::2] = np.cos(pos / (10000.0 ** (_2i / D_MODEL)))
    return np.concatenate([pe, pe[:CHUNK]], axis=0)


_PE_EXT_CONST = _positional_encoding_ext()


NBUF = 5
LOOK = 3


SEQ_PER_W = BATCH // NUM_WORKERS


XROWS_PER_W = ROWS_PER_W // CHUNK


def _body(x_hbm, pe_hbm, table_hbm, out_hbm, idx_v, pe_v, rows_v, gsem, ssem,
          isem):
    wid = lax.axis_index("s") * NUM_CORES + lax.axis_index("c")
    base = wid * ROWS_PER_W
    xrow0 = wid * XROWS_PER_W

    for r in range(XROWS_PER_W):
        pltpu.async_copy(x_hbm.at[xrow0 + r],
                         idx_v.at[pl.ds(r * CHUNK, CHUNK)], isem)
    pltpu.sync_copy(pe_hbm, pe_v)
    for r in range(XROWS_PER_W):
        pltpu.make_async_copy(x_hbm.at[xrow0],
                              idx_v.at[pl.ds(0, CHUNK)], isem).wait()

    def start_gather(c, b):
        pltpu.async_copy(
            table_hbm.at[idx_v.at[pl.ds(c * CHUNK, CHUNK)]],
            rows_v.at[b], gsem.at[b])

    def wait_gather(b):
        pltpu.make_async_copy(
            table_hbm.at[idx_v.at[pl.ds(0, CHUNK)]],
            rows_v.at[b], gsem.at[b]).wait()

    def start_store(c, b):
        pltpu.async_copy(
            rows_v.at[b], out_hbm.at[pl.ds(base + c * CHUNK, CHUNK)],
            ssem.at[b])

    def wait_store(b):
        pltpu.make_async_copy(
            rows_v.at[b], out_hbm.at[pl.ds(base, CHUNK)], ssem.at[b]).wait()

    for c in range(LOOK):
        start_gather(c, c)

    @pl.loop(0, NCHUNKS // NBUF)
    def _group(g):
        for b in range(NBUF):
            c = g * NBUF + b
            b2 = (b + LOOK) % NBUF

            @pl.when(c + LOOK < NCHUNKS)
            def _():
                @pl.when(c + LOOK >= NBUF)
                def _():
                    wait_store(b2)
                start_gather(c + LOOK, b2)

            wait_gather(b)
            s0 = lax.rem(c * CHUNK, SEQ_LEN)

            @pl.loop(0, CHUNK, unroll=4)
            def _row(i):
                for j in range(D_MODEL // 16):
                    sl = pl.ds(j * 16, 16)
                    rows_v[b, i, sl] += pe_v[s0 + i, sl]

            start_store(c, b)

    for b in range(NBUF):
        wait_store(b)


@jax.jit
def _embed(x2, pe_ext, table):
    kfn = pl.kernel(
        _body,
        name="embed_gather",
        out_type=jax.ShapeDtypeStruct((BT, D_MODEL), jnp.float32),
        mesh=plsc.VectorSubcoreMesh(core_axis_name="c", subcore_axis_name="s"),
        scratch_types=[
            pltpu.VMEM((ROWS_PER_W,), jnp.int32),
            pltpu.VMEM((PE_EXT, D_MODEL), jnp.float32),
            pltpu.VMEM((NBUF, CHUNK, D_MODEL), jnp.float32),
            pltpu.SemaphoreType.DMA((NBUF,)),
            pltpu.SemaphoreType.DMA((NBUF,)),
            pltpu.SemaphoreType.DMA,
        ],
        compiler_params=pltpu.CompilerParams(use_tc_tiling_on_sc=False),
    )
    return kfn(x2, pe_ext, table)


def kernel(x, table):
    pe_ext = jnp.asarray(_PE_EXT_CONST)
    x2 = x.astype(jnp.int32).reshape(BT // CHUNK, CHUNK)
    out = _embed(x2, pe_ext, table)
    return out.reshape(BATCH, SEQ_LEN, D_MODEL)


# speedup vs baseline: 1.1364x; 1.0648x over previous
import jax
import jax.numpy as jnp
import numpy as np
from jax import lax
from jax.experimental import pallas as pl
from jax.experimental.pallas import tpu as pltpu
from jax.experimental.pallas import tpu_sc as plsc

D_MODEL = 64
SEQ_LEN = 200
BATCH = 1024
XPAD = 256

NUM_CORES = 2
NUM_SUBCORES = 16
NUM_WORKERS = NUM_CORES * NUM_SUBCORES

SEQ_PER_W = BATCH // NUM_WORKERS
G0 = 128
G1 = SEQ_LEN - G0

NBUF = 4
LOOK = 2


def _positional_encoding() -> np.ndarray:
    pe = np.zeros((SEQ_LEN, D_MODEL), dtype=np.float32)
    pos = np.arange(0, SEQ_LEN, dtype=np.float32)[:, None]
    _2i = np.arange(0, D_MODEL, 2, dtype=np.float32)
    pe[:, 0::2] = np.sin(pos / (10000.0 ** (_2i / D_MODEL)))
    pe[:, 1::2] = np.cos(pos / (10000.0 ** (_2i / D_MODEL)))
    return pe


_PE_CONST = _positional_encoding()


def _body(x_hbm, pe_hbm, table_hbm, out_hbm, idx_v, pe_v, rows_v, gsem, ssem,
          isem):
    wid = lax.axis_index("s") * NUM_CORES + lax.axis_index("c")
    seq0 = wid * SEQ_PER_W

    for u in range(SEQ_PER_W):
        pltpu.async_copy(x_hbm.at[seq0 + u],
                         idx_v.at[pl.ds(u * XPAD, XPAD)], isem)
    pltpu.sync_copy(pe_hbm, pe_v)
    for u in range(SEQ_PER_W):
        pltpu.make_async_copy(x_hbm.at[seq0],
                              idx_v.at[pl.ds(0, XPAD)], isem).wait()

    def start_gather(u, b):
        off = u * XPAD
        pltpu.async_copy(
            table_hbm.at[idx_v.at[pl.ds(off, G0)]],
            rows_v.at[b, pl.ds(0, G0)], gsem.at[b])
        pltpu.async_copy(
            table_hbm.at[idx_v.at[pl.ds(off + G0, G1)]],
            rows_v.at[b, pl.ds(G0, G1)], gsem.at[b])

    def wait_gather(b):
        pltpu.make_async_copy(
            table_hbm.at[idx_v.at[pl.ds(0, G0)]],
            rows_v.at[b], gsem.at[b]).wait()

    def start_store(u, b):
        pltpu.async_copy(rows_v.at[b], out_hbm.at[seq0 + u], ssem.at[b])

    def wait_store(b):
        pltpu.make_async_copy(rows_v.at[b], out_hbm.at[seq0],
                              ssem.at[b]).wait()

    for u in range(LOOK):
        start_gather(u, u)

    @pl.loop(0, SEQ_PER_W // NBUF)
    def _group(g):
        for b in range(NBUF):
            u = g * NBUF + b
            b2 = (b + LOOK) % NBUF

            @pl.when(u + LOOK < SEQ_PER_W)
            def _():
                @pl.when(u + LOOK >= NBUF)
                def _():
                    wait_store(b2)
                start_gather(u + LOOK, b2)

            wait_gather(b)

            @pl.loop(0, SEQ_LEN, unroll=4)
            def _row(i):
                for j in range(D_MODEL // 16):
                    sl = pl.ds(j * 16, 16)
                    rows_v[b, i, sl] += pe_v[i, sl]

            start_store(u, b)

    for b in range(NBUF):
        wait_store(b)


@jax.jit
def _embed(x_p, pe, table):
    kfn = pl.kernel(
        _body,
        name="embed_gather",
        out_type=jax.ShapeDtypeStruct((BATCH, SEQ_LEN, D_MODEL), jnp.float32),
        mesh=plsc.VectorSubcoreMesh(core_axis_name="c", subcore_axis_name="s"),
        scratch_types=[
            pltpu.VMEM((SEQ_PER_W * XPAD,), jnp.int32),
            pltpu.VMEM((SEQ_LEN, D_MODEL), jnp.float32),
            pltpu.VMEM((NBUF, SEQ_LEN, D_MODEL), jnp.float32),
            pltpu.SemaphoreType.DMA((NBUF,)),
            pltpu.SemaphoreType.DMA((NBUF,)),
            pltpu.SemaphoreType.DMA,
        ],
        compiler_params=pltpu.CompilerParams(use_tc_tiling_on_sc=False),
    )
    return kfn(x_p, pe, table)


def kernel(x, table):
    pe = jnp.asarray(_PE_CONST)
    x_p = jnp.pad(x.astype(jnp.int32), ((0, 0), (0, XPAD - SEQ_LEN)))
    return _embed(x_p, pe, table)
